# baseline (speedup 1.0000x reference)
import jax
import jax.numpy as jnp
from jax.experimental import pallas as pl

BATCH = 32
SEQ_LEN = 8192
N_ALLELES = 4
D_MODEL = 64
S_TILE = 256


def _body(x_ref, a_ref, p_ref, o_ref):
    xb = x_ref[...].reshape(BATCH * S_TILE, N_ALLELES)
    emb = jax.lax.dot_general(
        xb, a_ref[...],
        dimension_numbers=(((1,), (0,)), ((), ())),
        preferred_element_type=jnp.float32,
    ).reshape(BATCH, S_TILE, D_MODEL)
    o_ref[...] = emb + p_ref[...][None, :, :]


def kernel(x, allele_embedding, position_table):
    grid = (SEQ_LEN // S_TILE,)
    return pl.pallas_call(
        _body,
        grid=grid,
        in_specs=[
            pl.BlockSpec((BATCH, S_TILE, N_ALLELES), lambda s: (0, s, 0)),
            pl.BlockSpec((N_ALLELES, D_MODEL), lambda s: (0, 0)),
            pl.BlockSpec((S_TILE, D_MODEL), lambda s: (s, 0)),
        ],
        out_specs=pl.BlockSpec((BATCH, S_TILE, D_MODEL), lambda s: (0, s, 0)),
        out_shape=jax.ShapeDtypeStruct((BATCH, SEQ_LEN, D_MODEL), jnp.float32),
    )(x, allele_embedding, position_table)
